# baseline (speedup 1.0000x reference)
import functools

import jax
import jax.numpy as jnp
from jax.experimental import pallas as pl
from jax.experimental.pallas import tpu as pltpu

_EPS = 1e-5
_CHUNK = 1024
_XSPLIT = 4
_G = 4


def _body(x0_ref, x1_ref, x2_ref, x3_ref, w1_ref, b1_ref, st_ref, wo_ref,
          bo_ref, g_ref, be_ref, out_ref, w_out_ref, ctx_s, *, t, h_heads):
    i = pl.program_id(0)
    x_refs = (x0_ref, x1_ref, x2_ref, x3_ref)
    c = x0_ref.shape[4]
    t_part = t // _XSPLIT

    prev = jnp.maximum(i - 1, 0)
    for g in range(_G):
        ctxp = ctx_s[g]
        acc = bo_ref[...]
        for h in range(h_heads):
            acc = acc + jnp.dot(ctxp[h:h + 1, :], wo_ref[h],
                                preferred_element_type=jnp.float32)
        mu = jnp.mean(acc, axis=1, keepdims=True)
        d = acc - mu
        var = jnp.mean(d * d, axis=1, keepdims=True)
        res = d * jax.lax.rsqrt(var + _EPS) * g_ref[...] + be_ref[...]
        out_ref[pl.ds(prev * _G + g, 1), :, :] = res.reshape(1, 1, c)

    w1 = w1_ref[...].astype(jnp.bfloat16)
    b1 = b1_ref[...]
    st_f32 = st_ref[...]
    st = st_f32.astype(jnp.bfloat16)

    m = jnp.sum(jnp.abs(st_f32), axis=1, keepdims=True)

    for g in range(_G):
        parts = []
        ctxs = [jnp.zeros((h_heads, c), dtype=jnp.float32) for _ in x_refs]
        for k in range(t // _CHUNK):
            q = k // (t_part // _CHUNK)
            x_ref = x_refs[q]
            j = k % (t_part // _CHUNK)
            xc_bf = x_ref[0, g, 0, j * _CHUNK:(j + 1) * _CHUNK, :].astype(
                jnp.bfloat16)
            hc = jnp.tanh(
                (jnp.dot(xc_bf, w1, preferred_element_type=jnp.float32)
                 + b1).astype(jnp.bfloat16))
            sc = jax.lax.dot_general(
                st, hc, (((1,), (1,)), ((), ())),
                preferred_element_type=jnp.float32)
            ec = jnp.exp(sc - m)
            parts.append(ec)
            ctxs[q] = ctxs[q] + jnp.dot(ec.astype(jnp.bfloat16), xc_bf,
                                        preferred_element_type=jnp.float32)
        e = jnp.concatenate(parts, axis=1)
        ctx = (ctxs[0] + ctxs[1]) + (ctxs[2] + ctxs[3])

        denom = jnp.sum(e, axis=1, keepdims=True)
        inv = 1.0 / denom
        w_out_ref[g] = e * inv
        ctx_s[g] = ctx * inv


def kernel(x, W1, b1, w2, Wo, bo, gamma, beta):
    b, t, c = x.shape
    h_heads, _, hd = W1.shape
    t_part = t // _XSPLIT
    n_pair = b // _G

    w1c = jnp.transpose(W1, (1, 0, 2)).reshape(c, h_heads * hd)
    b1f = b1.reshape(1, h_heads * hd)
    st = (jnp.eye(h_heads, dtype=x.dtype)[:, :, None] * w2[None, :, :]
          ).reshape(h_heads, h_heads * hd)
    wo_h = jnp.transpose(Wo.reshape(c, h_heads, c), (1, 2, 0))

    x5 = x.reshape(n_pair, _G, _XSPLIT, t_part, c)

    def x_spec(q):
        return pl.BlockSpec(
            (1, _G, 1, t_part, c),
            lambda i, _q=q: (jnp.minimum(i, n_pair - 1), 0, _q, 0, 0))

    body = functools.partial(_body, t=t, h_heads=h_heads)
    out3, wts = pl.pallas_call(
        body,
        grid=(n_pair + 1,),
        in_specs=[
            x_spec(0), x_spec(1), x_spec(2), x_spec(3),
            pl.BlockSpec((c, h_heads * hd), lambda i: (0, 0)),
            pl.BlockSpec((1, h_heads * hd), lambda i: (0, 0)),
            pl.BlockSpec((h_heads, h_heads * hd), lambda i: (0, 0)),
            pl.BlockSpec((h_heads, c, c), lambda i: (0, 0, 0)),
            pl.BlockSpec((1, c), lambda i: (0, 0)),
            pl.BlockSpec((1, c), lambda i: (0, 0)),
            pl.BlockSpec((1, c), lambda i: (0, 0)),
        ],
        out_specs=[
            pl.BlockSpec((b, 1, c), lambda i: (0, 0, 0)),
            pl.BlockSpec((_G, h_heads, t),
                         lambda i: (jnp.minimum(i, n_pair - 1), 0, 0)),
        ],
        out_shape=[
            jax.ShapeDtypeStruct((b, 1, c), jnp.float32),
            jax.ShapeDtypeStruct((b, h_heads, t), jnp.float32),
        ],
        scratch_shapes=[pltpu.VMEM((_G, h_heads, c), jnp.float32)],
        compiler_params=pltpu.CompilerParams(
            dimension_semantics=("arbitrary",),
        ),
    )(x5, x5, x5, x5,
      w1c, b1f, st, wo_h, bo.reshape(1, c), gamma.reshape(1, c),
      beta.reshape(1, c))
    return out3.reshape(b, c), wts


# speedup vs baseline: 1.0083x; 1.0083x over previous
import functools

import jax
import jax.numpy as jnp
from jax.experimental import pallas as pl
from jax.experimental.pallas import tpu as pltpu

_EPS = 1e-5
_CHUNK = 1024
_XSPLIT = 4
_G = 2


def _body(x0_ref, x1_ref, x2_ref, x3_ref, w1_ref, b1_ref, st_ref, wo_ref,
          bo_ref, g_ref, be_ref, out_ref, w_out_ref, ctx_s, *, t, h_heads):
    i = pl.program_id(0)
    x_refs = (x0_ref, x1_ref, x2_ref, x3_ref)
    c = x0_ref.shape[4]
    t_part = t // _XSPLIT

    prev = jnp.maximum(i - 1, 0)
    for g in range(_G):
        ctxp = ctx_s[g]
        acc = bo_ref[...]
        for h in range(h_heads):
            acc = acc + jnp.dot(ctxp[h:h + 1, :], wo_ref[h],
                                preferred_element_type=jnp.float32)
        mu = jnp.mean(acc, axis=1, keepdims=True)
        d = acc - mu
        var = jnp.mean(d * d, axis=1, keepdims=True)
        res = d * jax.lax.rsqrt(var + _EPS) * g_ref[...] + be_ref[...]
        out_ref[pl.ds(prev * _G + g, 1), :, :] = res.reshape(1, 1, c)

    w1 = w1_ref[...].astype(jnp.bfloat16)
    b1 = b1_ref[...]
    st_f32 = st_ref[...]
    st = st_f32.astype(jnp.bfloat16)

    m = jnp.sum(jnp.abs(st_f32), axis=1, keepdims=True)

    for g in range(_G):
        parts = []
        ctxs = [jnp.zeros((h_heads, c), dtype=jnp.float32) for _ in x_refs]
        for k in range(t // _CHUNK):
            q = k // (t_part // _CHUNK)
            x_ref = x_refs[q]
            j = k % (t_part // _CHUNK)
            xc_bf = x_ref[0, g, 0, j * _CHUNK:(j + 1) * _CHUNK, :].astype(
                jnp.bfloat16)
            hc = jnp.tanh(
                (jnp.dot(xc_bf, w1, preferred_element_type=jnp.float32)
                 + b1).astype(jnp.bfloat16))
            sc = jax.lax.dot_general(
                st, hc, (((1,), (1,)), ((), ())),
                preferred_element_type=jnp.float32)
            ec = jnp.exp(sc - m)
            parts.append(ec)
            ctxs[q] = ctxs[q] + jnp.dot(ec.astype(jnp.bfloat16), xc_bf,
                                        preferred_element_type=jnp.float32)
        e = jnp.concatenate(parts, axis=1)
        ctx = (ctxs[0] + ctxs[1]) + (ctxs[2] + ctxs[3])

        denom = jnp.sum(e, axis=1, keepdims=True)
        inv = 1.0 / denom
        w_out_ref[g] = e * inv
        ctx_s[g] = ctx * inv


def kernel(x, W1, b1, w2, Wo, bo, gamma, beta):
    b, t, c = x.shape
    h_heads, _, hd = W1.shape
    t_part = t // _XSPLIT
    n_pair = b // _G

    w1c = jnp.transpose(W1, (1, 0, 2)).reshape(c, h_heads * hd)
    b1f = b1.reshape(1, h_heads * hd)
    st = (jnp.eye(h_heads, dtype=x.dtype)[:, :, None] * w2[None, :, :]
          ).reshape(h_heads, h_heads * hd)
    wo_h = jnp.transpose(Wo.reshape(c, h_heads, c), (1, 2, 0))

    x5 = x.reshape(n_pair, _G, _XSPLIT, t_part, c)

    def x_spec(q):
        return pl.BlockSpec(
            (1, _G, 1, t_part, c),
            lambda i, _q=q: (jnp.minimum(i, n_pair - 1), 0, _q, 0, 0))

    body = functools.partial(_body, t=t, h_heads=h_heads)
    out3, wts = pl.pallas_call(
        body,
        grid=(n_pair + 1,),
        in_specs=[
            x_spec(0), x_spec(1), x_spec(2), x_spec(3),
            pl.BlockSpec((c, h_heads * hd), lambda i: (0, 0)),
            pl.BlockSpec((1, h_heads * hd), lambda i: (0, 0)),
            pl.BlockSpec((h_heads, h_heads * hd), lambda i: (0, 0)),
            pl.BlockSpec((h_heads, c, c), lambda i: (0, 0, 0)),
            pl.BlockSpec((1, c), lambda i: (0, 0)),
            pl.BlockSpec((1, c), lambda i: (0, 0)),
            pl.BlockSpec((1, c), lambda i: (0, 0)),
        ],
        out_specs=[
            pl.BlockSpec((b, 1, c), lambda i: (0, 0, 0)),
            pl.BlockSpec((_G, h_heads, t),
                         lambda i: (jnp.minimum(i, n_pair - 1), 0, 0)),
        ],
        out_shape=[
            jax.ShapeDtypeStruct((b, 1, c), jnp.float32),
            jax.ShapeDtypeStruct((b, h_heads, t), jnp.float32),
        ],
        scratch_shapes=[pltpu.VMEM((_G, h_heads, c), jnp.float32)],
        compiler_params=pltpu.CompilerParams(
            dimension_semantics=("arbitrary",),
        ),
    )(x5, x5, x5, x5,
      w1c, b1f, st, wo_h, bo.reshape(1, c), gamma.reshape(1, c),
      beta.reshape(1, c))
    return out3.reshape(b, c), wts
